# Initial kernel scaffold; baseline (speedup 1.0000x reference)
import jax
import jax.numpy as jnp
from jax.experimental import pallas as pl


def kernel(feat_p2, feat_p3, feat_p4, feat_p5, conv_w, conv_b, obj_w, obj_b, box_w, box_b, image_shapes):
    raise NotImplementedError("write your pallas kernel here")


# speedup vs baseline: 1.5734x; 1.5734x over previous
import functools

import numpy as np
import jax
import jax.numpy as jnp
from jax.experimental import pallas as pl
from jax.experimental.pallas import tpu as pltpu

_SIZES = ((32,), (64,), (128,), (256,))
_RATIOS = (0.5, 1.0, 2.0)
_STRIDES = (4, 8, 16, 32)
_PRE_NMS = 1000
_NMS_THR = 0.7
_MIN_SIZE = 16.0
_A = 3
_C = 256
_FEAT_SHAPES = ((128, 128), (64, 64), (32, 32), (16, 16))


def _np_anchors():
    cells = []
    for sizes in _SIZES:
        anchors = []
        for size in sizes:
            area = size ** 2
            for ar in _RATIOS:
                w = np.sqrt(area / ar)
                h = w * ar
                anchors.append([-w / 2, -h / 2, w / 2, h / 2])
        cells.append(np.asarray(anchors, np.float32))
    alls = []
    for lvl, (fh, fw) in enumerate(_FEAT_SHAPES):
        stride = _STRIDES[lvl]
        sx = np.arange(fw, dtype=np.float32) * stride
        sy = np.arange(fh, dtype=np.float32) * stride
        sy, sx = np.meshgrid(sy, sx, indexing='ij')
        shifts = np.stack([sx, sy, sx, sy], axis=2).reshape(-1, 4)
        a = (shifts[:, None, :] + cells[lvl][None, :, :]).reshape(-1, 4)
        alls.append(a)
    return np.concatenate(alls, axis=0)


_ANCHORS = _np_anchors()


def _conv_acc(x_ref, wt_ref, HW, W, nch):
    col = jax.lax.broadcasted_iota(jnp.int32, (HW, 1), 0) % W
    acc = jnp.zeros((HW, nch), jnp.float32)
    k = 0
    for kh in range(3):
        for kw in range(3):
            s = kh * W + kw
            xs = x_ref[pl.ds(s, HW), :]
            d = jax.lax.dot_general(
                xs, wt_ref[k], (((1,), (0,)), ((), ())),
                preferred_element_type=jnp.float32)
            if kw == 0:
                d = jnp.where(col != 0, d, 0.0)
            elif kw == 2:
                d = jnp.where(col != W - 1, d, 0.0)
            acc = acc + d
            k += 1
    return acc


def _conv_head_body(x_ref, wt_ref, cb_ref, wh_ref, bh_ref, o_ref, *, HW, W):
    acc = _conv_acc(x_ref, wt_ref, HW, W, _C)
    h = jnp.maximum(acc + cb_ref[...], 0.0)
    o_ref[...] = jax.lax.dot_general(
        h, wh_ref[...], (((1,), (0,)), ((), ())),
        preferred_element_type=jnp.float32) + bh_ref[...]


def _conv_only_body(x_ref, wt_ref, cb_ref, h_ref, *, HW, W, nch):
    acc = _conv_acc(x_ref, wt_ref, HW, W, nch)
    h_ref[...] = jnp.maximum(acc + cb_ref[...], 0.0)


def _head_body(h_ref, wh_ref, bh_ref, o_ref):
    o_ref[...] = jax.lax.dot_general(
        h_ref[...], wh_ref[...], (((1,), (0,)), ((), ())),
        preferred_element_type=jnp.float32) + bh_ref[...]


def _conv_head_level(xt_pad, w_taps, conv_b2, w_head, b_head2, HW, W,
                     interpret=False):
    if HW <= 4096:
        body = functools.partial(_conv_head_body, HW=HW, W=W)
        return pl.pallas_call(
            body,
            out_shape=jax.ShapeDtypeStruct((HW, 16), jnp.float32),
            interpret=interpret,
        )(xt_pad, w_taps, conv_b2, w_head, b_head2)
    nchunk = 2
    nch = _C // nchunk
    conv_body = functools.partial(_conv_only_body, HW=HW, W=W, nch=nch)
    h = pl.pallas_call(
        conv_body,
        grid=(nchunk,),
        in_specs=[
            pl.BlockSpec(xt_pad.shape, lambda j: (0, 0)),
            pl.BlockSpec((9, _C, nch), lambda j: (0, 0, j)),
            pl.BlockSpec((1, nch), lambda j: (0, j)),
        ],
        out_specs=pl.BlockSpec((HW, nch), lambda j: (0, j)),
        out_shape=jax.ShapeDtypeStruct((HW, _C), jnp.float32),
        interpret=interpret,
    )(xt_pad, w_taps, conv_b2)
    return pl.pallas_call(
        _head_body,
        out_shape=jax.ShapeDtypeStruct((HW, 16), jnp.float32),
        interpret=interpret,
    )(h, w_head, b_head2)


def _decode_boxes(anchors, deltas):
    w = anchors[:, 2] - anchors[:, 0]
    h = anchors[:, 3] - anchors[:, 1]
    cx = anchors[:, 0] + 0.5 * w
    cy = anchors[:, 1] + 0.5 * h
    dx, dy, dw, dh = deltas[:, 0], deltas[:, 1], deltas[:, 2], deltas[:, 3]
    pcx = dx * w + cx
    pcy = dy * h + cy
    pw = jnp.exp(dw) * w
    ph = jnp.exp(dh) * h
    return jnp.stack([pcx - 0.5 * pw, pcy - 0.5 * ph,
                      pcx + 0.5 * pw, pcy + 0.5 * ph], axis=1)


def _nms_mask(boxes, iou_thr):
    x1, y1, x2, y2 = boxes[:, 0], boxes[:, 1], boxes[:, 2], boxes[:, 3]
    areas = (x2 - x1) * (y2 - y1)
    xx1 = jnp.maximum(x1[:, None], x1[None, :])
    yy1 = jnp.maximum(y1[:, None], y1[None, :])
    xx2 = jnp.minimum(x2[:, None], x2[None, :])
    yy2 = jnp.minimum(y2[:, None], y2[None, :])
    inter = jnp.clip(xx2 - xx1, 0.0) * jnp.clip(yy2 - yy1, 0.0)
    iou = inter / (areas[:, None] + areas[None, :] - inter + 1e-9)
    n = boxes.shape[0]
    idx = jnp.arange(n)

    def body(i, keep):
        sup = keep[i] & (iou[i] > iou_thr) & (idx > i)
        return keep & (~sup)

    return jax.lax.fori_loop(0, n, body, jnp.ones((n,), dtype=bool))


def _kernel_impl(feat_p2, feat_p3, feat_p4, feat_p5, conv_w, conv_b, obj_w,
                 obj_b, box_w, box_b, image_shapes, interpret=False):
    feats = [feat_p2, feat_p3, feat_p4, feat_p5]
    w_taps = jnp.transpose(conv_w, (2, 3, 1, 0)).reshape(9, _C, _C)
    conv_b2 = conv_b.reshape(1, _C)
    wh = jnp.concatenate([obj_w.reshape(_A, _C), box_w.reshape(4 * _A, _C)],
                         axis=0)
    w_head = jnp.concatenate([wh, jnp.zeros((1, _C), jnp.float32)],
                             axis=0).T
    b_head2 = jnp.concatenate([obj_b, box_b,
                               jnp.zeros((1,), jnp.float32)]).reshape(1, 16)

    outs = []
    for lvl, f in enumerate(feats):
        H, W = _FEAT_SHAPES[lvl]
        HW = H * W
        xt = f.reshape(_C, HW).T
        pad = W + 1
        rpad = (-(HW + 2 * pad)) % 8
        xt_pad = jnp.pad(xt, ((pad, pad + rpad), (0, 0)))
        outs.append(_conv_head_level(xt_pad, w_taps, conv_b2, w_head, b_head2,
                                     HW, W, interpret=interpret))

    scores = jnp.concatenate([o[:, :_A].reshape(-1) for o in outs])
    deltas = jnp.concatenate([o[:, _A:_A + 4 * _A].reshape(-1, 4)
                              for o in outs], axis=0)

    img_h = image_shapes[0, 0].astype(jnp.float32)
    img_w = image_shapes[0, 1].astype(jnp.float32)
    anchors = jnp.asarray(_ANCHORS)
    props = _decode_boxes(anchors, deltas)
    props = jnp.stack([
        jnp.clip(props[:, 0], 0.0, img_w),
        jnp.clip(props[:, 1], 0.0, img_h),
        jnp.clip(props[:, 2], 0.0, img_w),
        jnp.clip(props[:, 3], 0.0, img_h),
    ], axis=1)
    w = props[:, 2] - props[:, 0]
    h = props[:, 3] - props[:, 1]
    valid = (w >= _MIN_SIZE) & (h >= _MIN_SIZE)
    scores = jnp.where(valid, scores, -1e9)
    top_scores, top_idx = jax.lax.top_k(scores, _PRE_NMS)
    top_props = props[top_idx]
    keep = _nms_mask(top_props, _NMS_THR)
    out_props = top_props * keep[:, None].astype(top_props.dtype)
    out_scores = jnp.where(keep, top_scores, 0.0)
    return out_props, out_scores


def kernel(feat_p2, feat_p3, feat_p4, feat_p5, conv_w, conv_b, obj_w, obj_b,
           box_w, box_b, image_shapes):
    return _kernel_impl(feat_p2, feat_p3, feat_p4, feat_p5, conv_w, conv_b,
                        obj_w, obj_b, box_w, box_b, image_shapes)


# speedup vs baseline: 5.8813x; 3.7380x over previous
import functools

import numpy as np
import jax
import jax.numpy as jnp
from jax.experimental import pallas as pl
from jax.experimental.pallas import tpu as pltpu

_SIZES = ((32,), (64,), (128,), (256,))
_RATIOS = (0.5, 1.0, 2.0)
_STRIDES = (4, 8, 16, 32)
_PRE_NMS = 1000
_NMS_THR = 0.7
_MIN_SIZE = 16.0
_A = 3
_C = 256
_FEAT_SHAPES = ((128, 128), (64, 64), (32, 32), (16, 16))


def _np_anchors():
    cells = []
    for sizes in _SIZES:
        anchors = []
        for size in sizes:
            area = size ** 2
            for ar in _RATIOS:
                w = np.sqrt(area / ar)
                h = w * ar
                anchors.append([-w / 2, -h / 2, w / 2, h / 2])
        cells.append(np.asarray(anchors, np.float32))
    alls = []
    for lvl, (fh, fw) in enumerate(_FEAT_SHAPES):
        stride = _STRIDES[lvl]
        sx = np.arange(fw, dtype=np.float32) * stride
        sy = np.arange(fh, dtype=np.float32) * stride
        sy, sx = np.meshgrid(sy, sx, indexing='ij')
        shifts = np.stack([sx, sy, sx, sy], axis=2).reshape(-1, 4)
        a = (shifts[:, None, :] + cells[lvl][None, :, :]).reshape(-1, 4)
        alls.append(a)
    return np.concatenate(alls, axis=0)


_ANCHORS = _np_anchors()
_ANCHORS_T = np.zeros((4, 65536), np.float32)
_ANCHORS_T[:, :_ANCHORS.shape[0]] = _ANCHORS.T


def _conv_acc(x_ref, wt_ref, HW, W, nch):
    col = jax.lax.broadcasted_iota(jnp.int32, (HW, 1), 0) % W
    acc = jnp.zeros((HW, nch), jnp.float32)
    k = 0
    for kh in range(3):
        for kw in range(3):
            s = kh * W + kw
            xs = x_ref[pl.ds(s, HW), :]
            d = jax.lax.dot_general(
                xs, wt_ref[k], (((1,), (0,)), ((), ())),
                preferred_element_type=jnp.float32)
            if kw == 0:
                d = jnp.where(col != 0, d, 0.0)
            elif kw == 2:
                d = jnp.where(col != W - 1, d, 0.0)
            acc = acc + d
            k += 1
    return acc


def _conv_head_body(x_ref, wt_ref, cb_ref, wh_ref, bh_ref, o_ref, *, HW, W):
    acc = _conv_acc(x_ref, wt_ref, HW, W, _C)
    h = jnp.maximum(acc + cb_ref[...], 0.0)
    o_ref[...] = jax.lax.dot_general(
        h, wh_ref[...], (((1,), (0,)), ((), ())),
        preferred_element_type=jnp.float32) + bh_ref[...]


def _conv_only_body(x_ref, wt_ref, cb_ref, h_ref, *, HW, W, nch):
    acc = _conv_acc(x_ref, wt_ref, HW, W, nch)
    h_ref[...] = jnp.maximum(acc + cb_ref[...], 0.0)


def _head_body(h_ref, wh_ref, bh_ref, o_ref):
    o_ref[...] = jax.lax.dot_general(
        h_ref[...], wh_ref[...], (((1,), (0,)), ((), ())),
        preferred_element_type=jnp.float32) + bh_ref[...]


def _conv_head_level(xt_pad, w_taps, conv_b2, w_head, b_head2, HW, W,
                     interpret=False):
    if HW <= 4096:
        body = functools.partial(_conv_head_body, HW=HW, W=W)
        return pl.pallas_call(
            body,
            out_shape=jax.ShapeDtypeStruct((HW, 16), jnp.float32),
            interpret=interpret,
        )(xt_pad, w_taps, conv_b2, w_head, b_head2)
    nchunk = 2
    nch = _C // nchunk
    conv_body = functools.partial(_conv_only_body, HW=HW, W=W, nch=nch)
    h = pl.pallas_call(
        conv_body,
        grid=(nchunk,),
        in_specs=[
            pl.BlockSpec(xt_pad.shape, lambda j: (0, 0)),
            pl.BlockSpec((9, _C, nch), lambda j: (0, 0, j)),
            pl.BlockSpec((1, nch), lambda j: (0, j)),
        ],
        out_specs=pl.BlockSpec((HW, nch), lambda j: (0, j)),
        out_shape=jax.ShapeDtypeStruct((HW, _C), jnp.float32),
        interpret=interpret,
    )(xt_pad, w_taps, conv_b2)
    return pl.pallas_call(
        _head_body,
        out_shape=jax.ShapeDtypeStruct((HW, 16), jnp.float32),
        interpret=interpret,
    )(h, w_head, b_head2)


_NCAND = 65536


def _decode_body(an_ref, de_ref, sc_ref, img_ref, pr_ref, so_ref):
    a0, a1 = an_ref[0:1, :], an_ref[1:2, :]
    a2, a3 = an_ref[2:3, :], an_ref[3:4, :]
    dx, dy = de_ref[0:1, :], de_ref[1:2, :]
    dw, dh = de_ref[2:3, :], de_ref[3:4, :]
    img_h = img_ref[0:1, 0:1]
    img_w = img_ref[0:1, 1:2]
    w = a2 - a0
    h = a3 - a1
    cx = a0 + 0.5 * w
    cy = a1 + 0.5 * h
    pcx = dx * w + cx
    pcy = dy * h + cy
    pw = jnp.exp(dw) * w
    ph = jnp.exp(dh) * h
    p0 = pcx - 0.5 * pw
    p1 = pcy - 0.5 * ph
    p2 = pcx + 0.5 * pw
    p3 = pcy + 0.5 * ph
    c0 = jnp.clip(p0, 0.0, img_w)
    c1 = jnp.clip(p1, 0.0, img_h)
    c2 = jnp.clip(p2, 0.0, img_w)
    c3 = jnp.clip(p3, 0.0, img_h)
    pr_ref[0:1, :] = c0
    pr_ref[1:2, :] = c1
    pr_ref[2:3, :] = c2
    pr_ref[3:4, :] = c3
    valid = ((c2 - c0) >= _MIN_SIZE) & ((c3 - c1) >= _MIN_SIZE)
    so_ref[...] = jnp.where(valid, sc_ref[...], -1e9)


def _decode_call(anchors_t, deltas_t, scores_r, img_r, interpret=False):
    return pl.pallas_call(
        _decode_body,
        out_shape=(jax.ShapeDtypeStruct((4, _NCAND), jnp.float32),
                   jax.ShapeDtypeStruct((1, _NCAND), jnp.float32)),
        interpret=interpret,
    )(anchors_t, deltas_t, scores_r, img_r)


_NSEL = 1024
_DEBUG_XLA_NMS = False


def _sort_nms_body(pay_ref, op_ref, os_ref, iou_ref):
    n = _NSEL
    pay = pay_ref[...]
    ridx = jax.lax.broadcasted_iota(jnp.int32, (n, 1), 0)
    lidx = jax.lax.broadcasted_iota(jnp.int32, (1, n), 1)
    real = ridx < _PRE_NMS
    score = jnp.where(real, pay[:, 4:5], -3.0e38)
    box = jnp.where(real, pay[:, 0:4], 0.0)

    b = jax.lax.bitcast_convert_type(score, jnp.int32)
    key = jnp.where(b >= 0, b, b ^ jnp.int32(0x7FFFFFFF))
    ident = (ridx == lidx).astype(jnp.float32)
    key_lo = (key & jnp.int32(0xFFFF)).astype(jnp.float32)
    key_hi = jnp.right_shift(key, 16).astype(jnp.float32)
    row_lo = jax.lax.dot_general(key_lo, ident, (((0,), (0,)), ((), ())),
                                 precision=jax.lax.Precision.HIGHEST,
                                 preferred_element_type=jnp.float32)
    row_hi = jax.lax.dot_general(key_hi, ident, (((0,), (0,)), ((), ())),
                                 precision=jax.lax.Precision.HIGHEST,
                                 preferred_element_type=jnp.float32)
    keyT = row_hi.astype(jnp.int32) * jnp.int32(65536) + row_lo.astype(jnp.int32)
    gt = (keyT > key)
    tie = (keyT == key) & (lidx < ridx)
    rank = jnp.sum(gt.astype(jnp.float32) + tie.astype(jnp.float32),
                   axis=1, keepdims=True)
    onehot = (rank == lidx.astype(jnp.float32)).astype(jnp.float32)

    pay2 = jnp.concatenate([box, score, jnp.zeros((n, 3), jnp.float32)],
                           axis=1)

    def permute_exact(oh, mat, transposed):
        bits = jax.lax.bitcast_convert_type(mat, jnp.int32)
        lo = (bits & jnp.int32(0xFFFF)).astype(jnp.float32)
        hi = jnp.right_shift(bits, 16).astype(jnp.float32)
        if transposed:
            dims = (((0,), (0,)), ((), ()))
            args = lambda half: (half, oh)
        else:
            dims = (((0,), (0,)), ((), ()))
            args = lambda half: (oh, half)
        lo_p = jax.lax.dot_general(*args(lo), dims,
                                   precision=jax.lax.Precision.HIGHEST,
                                   preferred_element_type=jnp.float32)
        hi_p = jax.lax.dot_general(*args(hi), dims,
                                   precision=jax.lax.Precision.HIGHEST,
                                   preferred_element_type=jnp.float32)
        out_bits = hi_p.astype(jnp.int32) * jnp.int32(65536) + \
            lo_p.astype(jnp.int32)
        return jax.lax.bitcast_convert_type(out_bits, jnp.float32)

    sorted_pay = permute_exact(onehot, pay2, transposed=False)
    sorted_t = permute_exact(onehot, pay2, transposed=True)
    x1, y1 = sorted_pay[:, 0:1], sorted_pay[:, 1:2]
    x2, y2 = sorted_pay[:, 2:3], sorted_pay[:, 3:4]
    x1t, y1t = sorted_t[0:1, :], sorted_t[1:2, :]
    x2t, y2t = sorted_t[2:3, :], sorted_t[3:4, :]
    areas = (x2 - x1) * (y2 - y1)
    areas_t = (x2t - x1t) * (y2t - y1t)
    xx1 = jnp.maximum(x1, x1t)
    yy1 = jnp.maximum(y1, y1t)
    xx2 = jnp.minimum(x2, x2t)
    yy2 = jnp.minimum(y2, y2t)
    inter = jnp.clip(xx2 - xx1, 0.0) * jnp.clip(yy2 - yy1, 0.0)
    iou_ref[...] = inter / (areas + areas_t - inter + 1e-9)

    def nms_step(i, keepf):
        j8 = pl.multiple_of((i // 8) * 8, 8)
        blk = iou_ref[pl.ds(j8, 8), :]
        rsel = jax.lax.broadcasted_iota(jnp.int32, (8, 1), 0) == (i - j8)
        row = jnp.sum(jnp.where(rsel, blk, 0.0), axis=0, keepdims=True)
        ki = jnp.sum(jnp.where(lidx == i, keepf, 0.0))
        supf = jnp.where((ki > 0.0) & (row > _NMS_THR) & (lidx > i), 1.0, 0.0)
        return keepf * (1.0 - supf)

    keep = jax.lax.fori_loop(0, _PRE_NMS, nms_step,
                             jnp.ones((1, n), dtype=jnp.float32))
    keep_col = jax.lax.dot_general(
        ident, keep, (((1,), (1,)), ((), ())),
        precision=jax.lax.Precision.HIGHEST,
        preferred_element_type=jnp.float32)
    if _DEBUG_XLA_NMS:
        op_ref[...] = sorted_pay[:, 0:4]
        os_ref[...] = sorted_pay[:, 4:5]
    else:
        op_ref[...] = sorted_pay[:, 0:4] * keep_col
        os_ref[...] = jnp.where(keep_col > 0.0, sorted_pay[:, 4:5], 0.0)


def _sort_nms(pay, interpret=False):
    return pl.pallas_call(
        _sort_nms_body,
        out_shape=(jax.ShapeDtypeStruct((_NSEL, 4), jnp.float32),
                   jax.ShapeDtypeStruct((_NSEL, 1), jnp.float32)),
        scratch_shapes=[pltpu.VMEM((_NSEL, _NSEL), jnp.float32)],
        interpret=interpret,
    )(pay)


def _decode_boxes(anchors, deltas):
    w = anchors[:, 2] - anchors[:, 0]
    h = anchors[:, 3] - anchors[:, 1]
    cx = anchors[:, 0] + 0.5 * w
    cy = anchors[:, 1] + 0.5 * h
    dx, dy, dw, dh = deltas[:, 0], deltas[:, 1], deltas[:, 2], deltas[:, 3]
    pcx = dx * w + cx
    pcy = dy * h + cy
    pw = jnp.exp(dw) * w
    ph = jnp.exp(dh) * h
    return jnp.stack([pcx - 0.5 * pw, pcy - 0.5 * ph,
                      pcx + 0.5 * pw, pcy + 0.5 * ph], axis=1)


def _nms_mask(boxes, iou_thr):
    x1, y1, x2, y2 = boxes[:, 0], boxes[:, 1], boxes[:, 2], boxes[:, 3]
    areas = (x2 - x1) * (y2 - y1)
    xx1 = jnp.maximum(x1[:, None], x1[None, :])
    yy1 = jnp.maximum(y1[:, None], y1[None, :])
    xx2 = jnp.minimum(x2[:, None], x2[None, :])
    yy2 = jnp.minimum(y2[:, None], y2[None, :])
    inter = jnp.clip(xx2 - xx1, 0.0) * jnp.clip(yy2 - yy1, 0.0)
    iou = inter / (areas[:, None] + areas[None, :] - inter + 1e-9)
    n = boxes.shape[0]
    idx = jnp.arange(n)

    def body(i, keep):
        sup = keep[i] & (iou[i] > iou_thr) & (idx > i)
        return keep & (~sup)

    return jax.lax.fori_loop(0, n, body, jnp.ones((n,), dtype=bool))


def _kernel_impl(feat_p2, feat_p3, feat_p4, feat_p5, conv_w, conv_b, obj_w,
                 obj_b, box_w, box_b, image_shapes, interpret=False):
    feats = [feat_p2, feat_p3, feat_p4, feat_p5]
    w_taps = jnp.transpose(conv_w, (2, 3, 1, 0)).reshape(9, _C, _C)
    conv_b2 = conv_b.reshape(1, _C)
    wh = jnp.concatenate([obj_w.reshape(_A, _C), box_w.reshape(4 * _A, _C)],
                         axis=0)
    w_head = jnp.concatenate([wh, jnp.zeros((1, _C), jnp.float32)],
                             axis=0).T
    b_head2 = jnp.concatenate([obj_b, box_b,
                               jnp.zeros((1,), jnp.float32)]).reshape(1, 16)

    outs = []
    for lvl, f in enumerate(feats):
        H, W = _FEAT_SHAPES[lvl]
        HW = H * W
        xt = f.reshape(_C, HW).T
        pad = W + 1
        rpad = (-(HW + 2 * pad)) % 8
        xt_pad = jnp.pad(xt, ((pad, pad + rpad), (0, 0)))
        outs.append(_conv_head_level(xt_pad, w_taps, conv_b2, w_head, b_head2,
                                     HW, W, interpret=interpret))

    scores = jnp.concatenate([o[:, :_A].reshape(-1) for o in outs])
    deltas = jnp.concatenate([o[:, _A:_A + 4 * _A].reshape(-1, 4)
                              for o in outs], axis=0)

    nreal = scores.shape[0]
    anchors_t = jnp.asarray(_ANCHORS_T)
    deltas_t = jnp.pad(deltas.T, ((0, 0), (0, _NCAND - nreal)))
    scores_r = jnp.pad(scores[None, :], ((0, 0), (0, _NCAND - nreal)))
    img_r = jnp.broadcast_to(
        image_shapes.astype(jnp.float32).reshape(1, 2), (1, 2))
    img_r = jnp.pad(img_r, ((0, 0), (0, 126)))
    props_t, scores_f = _decode_call(anchors_t, deltas_t, scores_r, img_r,
                                     interpret=interpret)
    props = props_t.T[:nreal]
    scores = scores_f[0, :nreal]
    top_scores, top_idx = jax.lax.top_k(scores, _PRE_NMS)
    top_props = props[top_idx]
    pay = jnp.concatenate(
        [top_props, top_scores[:, None], jnp.zeros((_PRE_NMS, 3), jnp.float32)],
        axis=1)
    pay = jnp.pad(pay, ((0, _NSEL - _PRE_NMS), (0, 0)))
    out_p, out_s = _sort_nms(pay, interpret=interpret)
    if _DEBUG_XLA_NMS:
        sboxes = out_p[:_PRE_NMS]
        sscores = out_s[:_PRE_NMS, 0]
        keep = _nms_mask(sboxes, _NMS_THR)
        return (sboxes * keep[:, None].astype(jnp.float32),
                jnp.where(keep, sscores, 0.0))
    return out_p[:_PRE_NMS], out_s[:_PRE_NMS, 0]


def kernel(feat_p2, feat_p3, feat_p4, feat_p5, conv_w, conv_b, obj_w, obj_b,
           box_w, box_b, image_shapes):
    return _kernel_impl(feat_p2, feat_p3, feat_p4, feat_p5, conv_w, conv_b,
                        obj_w, obj_b, box_w, box_b, image_shapes)
